# Initial kernel scaffold; baseline (speedup 1.0000x reference)
import jax
import jax.numpy as jnp
from jax.experimental import pallas as pl


def kernel(input_ids, wte, softprompt):
    raise NotImplementedError("write your pallas kernel here")


# speedup vs baseline: 3.9612x; 3.9612x over previous
import functools

import jax
import jax.numpy as jnp
from jax import lax
from jax.experimental import pallas as pl
from jax.experimental.pallas import tpu as pltpu
from jax.experimental.pallas import tpu_sc as plsc

VOCAB = 100000
D = 128
P = 20
B = 1024
S = 220
T = S - P
HALF = T // 2

NC = 2
NS = 16
NW = NC * NS
BPW = B // NW

_MESH = plsc.VectorSubcoreMesh(
    core_axis_name="c", subcore_axis_name="s", num_cores=NC, num_subcores=NS
)


def _body(wte_hbm, ids_hbm, sp_hbm, out_hbm, idx_v, obuf, sem):
    wid = lax.axis_index("s") * NC + lax.axis_index("c")
    pltpu.sync_copy(sp_hbm, obuf.at[pl.ds(0, P)])

    @pl.loop(0, BPW)
    def per_batch(j):
        b = wid * BPW + j
        pltpu.sync_copy(ids_hbm.at[b], idx_v)
        c0 = pltpu.async_copy(
            wte_hbm.at[idx_v.at[0]], obuf.at[pl.ds(P, HALF)], sem
        )
        c1 = pltpu.async_copy(
            wte_hbm.at[idx_v.at[1]], obuf.at[pl.ds(P + HALF, HALF)], sem
        )
        c0.wait()
        c1.wait()
        pltpu.sync_copy(obuf, out_hbm.at[b])


_sc_call = functools.partial(
    pl.kernel,
    out_type=jax.ShapeDtypeStruct((B, S, D), jnp.float32),
    mesh=_MESH,
    scratch_types=[
        pltpu.VMEM((2, HALF), jnp.int32),
        pltpu.VMEM((S, D), jnp.float32),
        pltpu.SemaphoreType.DMA,
    ],
)(_body)


@jax.jit
def kernel(input_ids, wte, softprompt):
    ids3 = input_ids[:, P:].reshape(B, 2, HALF).astype(jnp.int32)
    return _sc_call(wte, ids3, softprompt)


# speedup vs baseline: 4.6895x; 1.1839x over previous
import functools

import jax
import jax.numpy as jnp
from jax import lax
from jax.experimental import pallas as pl
from jax.experimental.pallas import tpu as pltpu
from jax.experimental.pallas import tpu_sc as plsc

VOCAB = 100000
D = 128
P = 20
B = 1024
S = 220
T = S - P
HALF = T // 2

NC = 2
NS = 16
NW = NC * NS
BPW = B // NW
NBUF = 4

_MESH = plsc.VectorSubcoreMesh(
    core_axis_name="c", subcore_axis_name="s", num_cores=NC, num_subcores=NS
)


def _body(wte_hbm, ids_hbm, sp_hbm, out_hbm, idx_v, obuf, sem_g, sem_st):
    wid = lax.axis_index("s") * NC + lax.axis_index("c")

    pltpu.sync_copy(ids_hbm.at[wid], idx_v)
    for s in range(NBUF):
        pltpu.sync_copy(sp_hbm, obuf.at[s, pl.ds(0, P)])

    def g_start(j, s):
        pltpu.async_copy(
            wte_hbm.at[idx_v.at[j, 0]], obuf.at[s, pl.ds(P, HALF)], sem_g.at[s]
        )
        pltpu.async_copy(
            wte_hbm.at[idx_v.at[j, 1]],
            obuf.at[s, pl.ds(P + HALF, HALF)],
            sem_g.at[s],
        )

    def g_wait(s):
        pltpu.make_async_copy(
            wte_hbm.at[idx_v.at[0, 0]], obuf.at[s, pl.ds(P, HALF)], sem_g.at[s]
        ).wait()
        pltpu.make_async_copy(
            wte_hbm.at[idx_v.at[0, 1]],
            obuf.at[s, pl.ds(P + HALF, HALF)],
            sem_g.at[s],
        ).wait()

    def st_start(j, s):
        pltpu.async_copy(obuf.at[s], out_hbm.at[wid * BPW + j], sem_st.at[s])

    def st_wait(s):
        pltpu.make_async_copy(obuf.at[s], out_hbm.at[0], sem_st.at[s]).wait()

    @pl.loop(0, BPW)
    def per_batch(j):
        s = lax.rem(j, NBUF)

        @pl.when(j >= NBUF)
        def _free_slot():
            st_wait(s)

        g_start(j, s)

        @pl.when(j >= 1)
        def _finish_prev():
            sp = lax.rem(j - 1, NBUF)
            g_wait(sp)
            st_start(j - 1, sp)

    last = BPW - 1
    g_wait(last % NBUF)
    st_start(last, last % NBUF)
    for s in range(NBUF):
        st_wait(s)


_sc_call = functools.partial(
    pl.kernel,
    out_type=jax.ShapeDtypeStruct((B, S, D), jnp.float32),
    mesh=_MESH,
    scratch_types=[
        pltpu.VMEM((BPW, 2, HALF), jnp.int32),
        pltpu.VMEM((NBUF, S, D), jnp.float32),
        pltpu.SemaphoreType.DMA((NBUF,)),
        pltpu.SemaphoreType.DMA((NBUF,)),
    ],
)(_body)


@jax.jit
def kernel(input_ids, wte, softprompt):
    ids4 = input_ids[:, P:].reshape(NW, BPW, 2, HALF).astype(jnp.int32)
    return _sc_call(wte, ids4, softprompt)
